# baseline (speedup 1.0000x reference)
import functools

import jax
import jax.numpy as jnp
from jax import lax
from jax.experimental import pallas as pl
from jax.experimental.pallas import tpu as pltpu
from jax.experimental.pallas import tpu_sc as plsc

_B = 8192
_P = 10000
_ONES = 9904
_FIX = 20000
_ZEROS = 10000
_TLEN = 20112
_NW = 32
_BPW = _B // _NW
_G = 16


def _sc_body(n_hbm, tmpl_hbm, out_hbm, nv, tmpl, sem):
    wid = lax.axis_index("s") * 2 + lax.axis_index("c")
    base = wid * _BPW

    pltpu.sync_copy(tmpl_hbm, tmpl)
    pltpu.sync_copy(n_hbm.at[pl.ds(base, _BPW)], nv)

    def bulk_group(g, carry):
        @pl.when(g >= 1)
        def _wait_prev_group():
            def drain_one(j, c):
                pltpu.make_async_copy(
                    tmpl.at[pl.ds(0, _P)], out_hbm.at[pl.ds(0, _P)], sem
                ).wait()
                return c

            lax.fori_loop(0, _G, drain_one, 0)

        ns = nv[pl.ds(g * _G, _G)]
        offs = _ONES - 100 * ns - 4 * (ns & 1)
        for k in range(_G):
            src = pl.multiple_of(offs[k], 8)
            dst = pl.multiple_of((base + g * _G + k) * _P, 8)
            pltpu.make_async_copy(
                tmpl.at[pl.ds(src, _P)], out_hbm.at[pl.ds(dst, _P)], sem
            ).start()
        return carry

    lax.fori_loop(0, _BPW // _G, bulk_group, 0)

    def bulk_drain(j, carry):
        pltpu.make_async_copy(
            tmpl.at[pl.ds(0, _P)], out_hbm.at[pl.ds(0, _P)], sem
        ).wait()
        return carry

    lax.fori_loop(0, _G, bulk_drain, 0)

    def fix_group(g, carry):
        @pl.when(g >= 1)
        def _wait_prev_group():
            def drain_one(j, c):
                pltpu.make_async_copy(
                    tmpl.at[pl.ds(0, 104)], out_hbm.at[pl.ds(0, 104)], sem
                ).wait()
                return c

            lax.fori_loop(0, _G, drain_one, 0)

        ns = nv[pl.ds(g * _G, _G)]
        odd = ns & 1
        srcs = _ZEROS + (_FIX - _ZEROS) * odd
        dsts = 100 * ns - 4 * odd
        for k in range(_G):
            src = pl.multiple_of(srcs[k], 8)
            dst = pl.multiple_of((base + g * _G + k) * _P + dsts[k], 8)
            pltpu.make_async_copy(
                tmpl.at[pl.ds(src, 104)], out_hbm.at[pl.ds(dst, 104)], sem
            ).start()
        return carry

    lax.fori_loop(0, _BPW // _G, fix_group, 0)

    def fix_drain(j, carry):
        pltpu.make_async_copy(
            tmpl.at[pl.ds(0, 104)], out_hbm.at[pl.ds(0, 104)], sem
        ).wait()
        return carry

    lax.fori_loop(0, _G, fix_drain, 0)


_sc_fill = functools.partial(
    pl.kernel,
    out_type=jax.ShapeDtypeStruct((_B * _P,), jnp.float32),
    mesh=plsc.VectorSubcoreMesh(core_axis_name="c", subcore_axis_name="s"),
    scratch_types=[
        pltpu.VMEM((_BPW,), jnp.int32),
        pltpu.VMEM((_TLEN,), jnp.float32),
        pltpu.SemaphoreType.DMA,
    ],
)(_sc_body)


def kernel(tensor_span):
    b = tensor_span.shape[0]
    n = tensor_span[:, 0]
    idx = jnp.arange(_TLEN, dtype=jnp.int32)
    tmpl = ((idx < _ONES) | ((idx >= _FIX) & (idx < _FIX + 4))).astype(
        jnp.float32
    )
    out = _sc_fill(n, tmpl)
    return out.reshape(b, 100, 100)


# speedup vs baseline: 1.0010x; 1.0010x over previous
import functools

import jax
import jax.numpy as jnp
from jax import lax
from jax.experimental import pallas as pl
from jax.experimental.pallas import tpu as pltpu
from jax.experimental.pallas import tpu_sc as plsc

_B = 8192
_P = 10000
_ONES = 9904
_FIX = 20000
_ZEROS = 10000
_TLEN = 20112
_NW = 32
_BPW = _B // _NW
_G = 16


def _sc_body(n_hbm, tmpl_hbm, out_hbm, nv, tmpl, sem):
    wid = lax.axis_index("s") * 2 + lax.axis_index("c")
    base = wid * _BPW

    pltpu.sync_copy(tmpl_hbm, tmpl)
    pltpu.sync_copy(n_hbm.at[pl.ds(base, _BPW)], nv)

    def bulk_group(g, carry):
        ns = nv[pl.ds(g * _G, _G)]
        offs = _ONES - 100 * ns - 4 * (ns & 1)
        for k in range(_G):
            src = pl.multiple_of(offs[k], 8)
            dst = pl.multiple_of((base + g * _G + k) * _P, 8)
            pltpu.make_async_copy(
                tmpl.at[pl.ds(src, _P)], out_hbm.at[pl.ds(dst, _P)], sem
            ).start()
        return carry

    lax.fori_loop(0, _BPW // _G, bulk_group, 0)

    def bulk_drain(j, carry):
        pltpu.make_async_copy(
            tmpl.at[pl.ds(0, _P)], out_hbm.at[pl.ds(0, _P)], sem
        ).wait()
        return carry

    lax.fori_loop(0, _BPW, bulk_drain, 0)

    def fix_group(g, carry):
        ns = nv[pl.ds(g * _G, _G)]
        odd = ns & 1
        srcs = _ZEROS + (_FIX - _ZEROS) * odd
        dsts = 100 * ns - 4 * odd
        for k in range(_G):
            src = pl.multiple_of(srcs[k], 8)
            dst = pl.multiple_of((base + g * _G + k) * _P + dsts[k], 8)
            pltpu.make_async_copy(
                tmpl.at[pl.ds(src, 104)], out_hbm.at[pl.ds(dst, 104)], sem
            ).start()
        return carry

    lax.fori_loop(0, _BPW // _G, fix_group, 0)

    def fix_drain(j, carry):
        pltpu.make_async_copy(
            tmpl.at[pl.ds(0, 104)], out_hbm.at[pl.ds(0, 104)], sem
        ).wait()
        return carry

    lax.fori_loop(0, _BPW, fix_drain, 0)


_sc_fill = functools.partial(
    pl.kernel,
    out_type=jax.ShapeDtypeStruct((_B * _P,), jnp.float32),
    mesh=plsc.VectorSubcoreMesh(core_axis_name="c", subcore_axis_name="s"),
    scratch_types=[
        pltpu.VMEM((_BPW,), jnp.int32),
        pltpu.VMEM((_TLEN,), jnp.float32),
        pltpu.SemaphoreType.DMA,
    ],
)(_sc_body)


def kernel(tensor_span):
    b = tensor_span.shape[0]
    n = tensor_span[:, 0]
    idx = jnp.arange(_TLEN, dtype=jnp.int32)
    tmpl = ((idx < _ONES) | ((idx >= _FIX) & (idx < _FIX + 4))).astype(
        jnp.float32
    )
    out = _sc_fill(n, tmpl)
    return out.reshape(b, 100, 100)


# speedup vs baseline: 1.2177x; 1.2165x over previous
import functools

import jax
import jax.numpy as jnp
from jax import lax
from jax.experimental import pallas as pl
from jax.experimental.pallas import tpu as pltpu
from jax.experimental.pallas import tpu_sc as plsc

_B = 8192
_NW = 32
_BPW = _B // _NW
_G = 16


def _sc_body(n_hbm, tmpl_hbm, out_hbm, nv, tmpl, sem):
    sid = lax.axis_index("s")
    wid = sid * 2 + lax.axis_index("c")
    base = wid * _BPW

    @pl.when(sid == 0)
    def _stage_template():
        pltpu.sync_copy(tmpl_hbm, tmpl)

    plsc.subcore_barrier()
    pltpu.sync_copy(n_hbm.at[pl.ds(base, _BPW)], nv)

    def issue_group(g, carry):
        ns = nv[pl.ds(g * _G, _G)]
        offs = 99 - ns
        for k in range(_G):
            pltpu.make_async_copy(
                tmpl.at[pl.ds(offs[k], 100), :],
                out_hbm.at[base + g * _G + k],
                sem,
            ).start()
        return carry

    lax.fori_loop(0, _BPW // _G, issue_group, 0)

    def drain(j, carry):
        pltpu.make_async_copy(
            tmpl.at[pl.ds(0, 100), :], out_hbm.at[0], sem
        ).wait()
        return carry

    lax.fori_loop(0, _BPW, drain, 0)


_sc_fill = functools.partial(
    pl.kernel,
    out_type=jax.ShapeDtypeStruct((_B, 100, 100), jnp.float32),
    mesh=plsc.VectorSubcoreMesh(core_axis_name="c", subcore_axis_name="s"),
    scratch_types=[
        pltpu.VMEM((_BPW,), jnp.int32),
        pltpu.VMEM_SHARED((199, 100), jnp.float32),
        pltpu.SemaphoreType.DMA,
    ],
)(_sc_body)


def kernel(tensor_span):
    b = tensor_span.shape[0]
    n = tensor_span[:, 0]
    tmpl = (jnp.arange(199, dtype=jnp.int32)[:, None] < 99).astype(jnp.float32)
    tmpl = jnp.broadcast_to(tmpl, (199, 100))
    return _sc_fill(n, tmpl)
